# Initial kernel scaffold; baseline (speedup 1.0000x reference)
import jax
import jax.numpy as jnp
from jax.experimental import pallas as pl


def kernel(q, k, v, edge_index, Wq, Wk, Wv, Wo, bo, Wg, bg):
    raise NotImplementedError("write your pallas kernel here")


# speedup vs baseline: 6948.3769x; 6948.3769x over previous
import jax
import jax.numpy as jnp
from jax.experimental import pallas as pl
from jax.experimental.pallas import tpu as pltpu

E = 16000
N = 2000
D = 256
H = 8
DH = 32
TD = H * DH
NORM = DH ** -0.5

BQ = 128
BK = 256
EK = 16128
NBQ = (2 * E) // BQ

_INTERPRET = False


def _proj_q_body(x_ref, w_ref, o_ref):
    o_ref[...] = jax.lax.dot_general(
        x_ref[...], w_ref[...], (((1,), (1,)), ((), ())),
        preferred_element_type=jnp.float32) * NORM


def _proj_kv_body(k_ref, v_ref, wk_ref, wv_ref, kh_ref, vh_ref):
    kh_ref[...] = jax.lax.dot_general(
        k_ref[...], wk_ref[...], (((1,), (1,)), ((), ())),
        preferred_element_type=jnp.float32)
    vh_ref[...] = jax.lax.dot_general(
        v_ref[...], wv_ref[...], (((1,), (1,)), ((), ())),
        preferred_element_type=jnp.float32)


def _attn_body(starts_ref, ends_ref, qh_ref, qn_ref, kh_ref, vh_ref, kn_ref,
               num_ref, den_ref):
    b = pl.program_id(0)
    c0 = starts_ref[b] // BK
    c1 = (ends_ref[b] + BK - 1) // BK
    qh = qh_ref[...]
    qn = qn_ref[...]

    def body(c, carry):
        num, den = carry
        off = c * BK
        kh = kh_ref[pl.ds(off, BK), :]
        vh = vh_ref[pl.ds(off, BK), :]
        kn = kn_ref[pl.ds(off, BK)]
        mask = (qn[:, None] == kn[None, :]).astype(jnp.float32)
        nums, dens = [], []
        for h in range(H):
            s = jax.lax.dot_general(
                qh[:, h * DH:(h + 1) * DH], kh[:, h * DH:(h + 1) * DH],
                (((1,), (1,)), ((), ())), preferred_element_type=jnp.float32)
            p = jnp.exp(s) * mask
            dens.append(jnp.sum(p, axis=1, keepdims=True))
            nums.append(jax.lax.dot_general(
                p, vh[:, h * DH:(h + 1) * DH],
                (((1,), (0,)), ((), ())), preferred_element_type=jnp.float32))
        return (num + jnp.concatenate(nums, axis=1),
                den + jnp.concatenate(dens, axis=1))

    num0 = jnp.zeros((BQ, TD), jnp.float32)
    den0 = jnp.zeros((BQ, H), jnp.float32)
    num, den = jax.lax.fori_loop(c0, c1, body, (num0, den0))
    num_ref[...] = num
    den_ref[...] = den


def _combine_body(q_ref, na_ref, nb_ref, da_ref, db_ref, m_ref,
                  wg_ref, bg_ref, wo_ref, bo_ref, out_ref):
    m = m_ref[...][:, None]
    num = na_ref[...] + nb_ref[...] * m
    den = da_ref[...] + db_ref[...] * m
    parts = [num[:, h * DH:(h + 1) * DH] / den[:, h:h + 1] for h in range(H)]
    o = jnp.concatenate(parts, axis=1)
    g = jax.lax.dot_general(
        q_ref[...], wg_ref[...], (((1,), (1,)), ((), ())),
        preferred_element_type=jnp.float32) + bg_ref[...]
    o = o / (1.0 + jnp.exp(-g))
    out_ref[...] = jax.lax.dot_general(
        o, wo_ref[...], (((1,), (1,)), ((), ())),
        preferred_element_type=jnp.float32) + bo_ref[...]


def kernel(q, k, v, edge_index, Wq, Wk, Wv, Wo, bo, Wg, bg):
    f32 = jnp.float32
    src = edge_index[:, 0]
    dst = edge_index[:, 1]
    perm_s = jnp.argsort(src)
    perm_d = jnp.argsort(dst)
    srcs = jnp.take(src, perm_s)
    dstd = jnp.take(dst, perm_d)
    counts = jnp.bincount(src, length=N)
    off = jnp.concatenate([jnp.zeros((1,), jnp.int32),
                           jnp.cumsum(counts).astype(jnp.int32)])

    qcat = jnp.concatenate([jnp.take(q, perm_s, axis=0),
                            jnp.take(q, perm_d, axis=0)], axis=0)
    qnode_i = jnp.concatenate([srcs, dstd]).astype(jnp.int32)
    kps = jnp.concatenate([jnp.take(k, perm_s, axis=0),
                           jnp.zeros((EK - E, D), f32)], axis=0)
    vps = jnp.concatenate([jnp.take(v, perm_s, axis=0),
                           jnp.zeros((EK - E, D), f32)], axis=0)
    kn = jnp.concatenate([srcs.astype(f32), jnp.full((EK - E,), -1.0, f32)])

    starts = jnp.take(off, qnode_i[0::BQ]).astype(jnp.int32)
    ends = jnp.take(off, qnode_i[BQ - 1::BQ] + 1).astype(jnp.int32)
    qn = qnode_i.astype(f32)

    qh = pl.pallas_call(
        _proj_q_body,
        grid=(NBQ,),
        in_specs=[pl.BlockSpec((BQ, D), lambda b: (b, 0)),
                  pl.BlockSpec((TD, D), lambda b: (0, 0))],
        out_specs=pl.BlockSpec((BQ, TD), lambda b: (b, 0)),
        out_shape=jax.ShapeDtypeStruct((2 * E, TD), f32),
        interpret=_INTERPRET,
    )(qcat, Wq)

    khs, vhs = pl.pallas_call(
        _proj_kv_body,
        grid=(EK // BQ,),
        in_specs=[pl.BlockSpec((BQ, D), lambda b: (b, 0)),
                  pl.BlockSpec((BQ, D), lambda b: (b, 0)),
                  pl.BlockSpec((TD, D), lambda b: (0, 0)),
                  pl.BlockSpec((TD, D), lambda b: (0, 0))],
        out_specs=[pl.BlockSpec((BQ, TD), lambda b: (b, 0)),
                   pl.BlockSpec((BQ, TD), lambda b: (b, 0))],
        out_shape=[jax.ShapeDtypeStruct((EK, TD), f32),
                   jax.ShapeDtypeStruct((EK, TD), f32)],
        interpret=_INTERPRET,
    )(kps, vps, Wk, Wv)

    numer, den = pl.pallas_call(
        _attn_body,
        grid=(NBQ,),
        in_specs=[pl.BlockSpec(memory_space=pltpu.SMEM),
                  pl.BlockSpec(memory_space=pltpu.SMEM),
                  pl.BlockSpec((BQ, TD), lambda b: (b, 0)),
                  pl.BlockSpec((BQ,), lambda b: (b,)),
                  pl.BlockSpec((EK, TD), lambda b: (0, 0)),
                  pl.BlockSpec((EK, TD), lambda b: (0, 0)),
                  pl.BlockSpec((EK,), lambda b: (0,))],
        out_specs=[pl.BlockSpec((BQ, TD), lambda b: (b, 0)),
                   pl.BlockSpec((BQ, H), lambda b: (b, 0))],
        out_shape=[jax.ShapeDtypeStruct((2 * E, TD), f32),
                   jax.ShapeDtypeStruct((2 * E, H), f32)],
        interpret=_INTERPRET,
    )(starts, ends, qh, qn, khs, vhs, kn)

    inv_s = jnp.zeros((E,), jnp.int32).at[perm_s].set(jnp.arange(E, dtype=jnp.int32))
    inv_d = jnp.zeros((E,), jnp.int32).at[perm_d].set(jnp.arange(E, dtype=jnp.int32))
    na = jnp.take(numer[:E], inv_s, axis=0)
    da = jnp.take(den[:E], inv_s, axis=0)
    nb = jnp.take(numer[E:], inv_d, axis=0)
    db = jnp.take(den[E:], inv_d, axis=0)
    m = (src != dst).astype(f32)

    out = pl.pallas_call(
        _combine_body,
        grid=(E // BQ,),
        in_specs=[pl.BlockSpec((BQ, D), lambda b: (b, 0)),
                  pl.BlockSpec((BQ, TD), lambda b: (b, 0)),
                  pl.BlockSpec((BQ, TD), lambda b: (b, 0)),
                  pl.BlockSpec((BQ, H), lambda b: (b, 0)),
                  pl.BlockSpec((BQ, H), lambda b: (b, 0)),
                  pl.BlockSpec((BQ,), lambda b: (b,)),
                  pl.BlockSpec((TD, D), lambda b: (0, 0)),
                  pl.BlockSpec((1, TD), lambda b: (0, 0)),
                  pl.BlockSpec((D, TD), lambda b: (0, 0)),
                  pl.BlockSpec((1, D), lambda b: (0, 0))],
        out_specs=pl.BlockSpec((BQ, D), lambda b: (b, 0)),
        out_shape=jax.ShapeDtypeStruct((E, D), f32),
        interpret=_INTERPRET,
    )(q, na, nb, da, db, m, Wg, bg.reshape(1, TD), Wo, bo.reshape(1, D))
    return out


# speedup vs baseline: 7552.9885x; 1.0870x over previous
import functools

import jax
import jax.numpy as jnp
from jax import lax
from jax.experimental import pallas as pl
from jax.experimental.pallas import tpu as pltpu
from jax.experimental.pallas import tpu_sc as plsc

E = 16000
N = 2000
D = 256
H = 8
DH = 32
TD = H * DH
NORM = DH ** -0.5

BQ = 128
BK = 256
EK = 16128
NBQ = (2 * E) // BQ

_INTERPRET = False

_NC, _NS = 2, 16
_NW = _NC * _NS


def _sc_gather_rows(table, idx, rows_chunk):
    B = idx.shape[0]
    C = table.shape[1]
    b_per_w = B // _NW
    nchunks = b_per_w // rows_chunk
    mesh = plsc.VectorSubcoreMesh(core_axis_name="c", subcore_axis_name="s")

    @functools.partial(
        pl.kernel, mesh=mesh,
        out_type=jax.ShapeDtypeStruct((B, C), jnp.float32),
        scratch_types=[pltpu.VMEM((rows_chunk,), jnp.int32),
                       pltpu.VMEM((rows_chunk, C), jnp.float32),
                       pltpu.SemaphoreType.DMA],
    )
    def kern(table_hbm, idx_hbm, out_hbm, idx_v, rows_v, sem):
        wid = lax.axis_index("s") * _NC + lax.axis_index("c")
        base = wid * b_per_w
        for ci in range(nchunks):
            o = base + ci * rows_chunk
            pltpu.sync_copy(idx_hbm.at[pl.ds(o, rows_chunk)], idx_v)
            pltpu.async_copy(table_hbm.at[idx_v], rows_v, sem).wait()
            pltpu.sync_copy(rows_v, out_hbm.at[pl.ds(o, rows_chunk)])

    return kern(table, idx)


def _proj_q_body(x_ref, w_ref, o_ref):
    o_ref[...] = jax.lax.dot_general(
        x_ref[...], w_ref[...], (((1,), (1,)), ((), ())),
        preferred_element_type=jnp.float32) * NORM


def _proj_kv_body(k_ref, v_ref, wk_ref, wv_ref, kh_ref, vh_ref):
    kh_ref[...] = jax.lax.dot_general(
        k_ref[...], wk_ref[...], (((1,), (1,)), ((), ())),
        preferred_element_type=jnp.float32)
    vh_ref[...] = jax.lax.dot_general(
        v_ref[...], wv_ref[...], (((1,), (1,)), ((), ())),
        preferred_element_type=jnp.float32)


def _attn_body(starts_ref, ends_ref, qh_ref, qn_ref, kh_ref, vh_ref, kn_ref,
               num_ref, den_ref):
    b = pl.program_id(0)
    c0 = starts_ref[b] // BK
    c1 = (ends_ref[b] + BK - 1) // BK
    qh = qh_ref[...]
    qn = qn_ref[...]

    def body(c, carry):
        num, den = carry
        off = c * BK
        kh = kh_ref[pl.ds(off, BK), :]
        vh = vh_ref[pl.ds(off, BK), :]
        kn = kn_ref[pl.ds(off, BK)]
        mask = (qn[:, None] == kn[None, :]).astype(jnp.float32)
        nums, dens = [], []
        for h in range(H):
            s = jax.lax.dot_general(
                qh[:, h * DH:(h + 1) * DH], kh[:, h * DH:(h + 1) * DH],
                (((1,), (1,)), ((), ())), preferred_element_type=jnp.float32)
            p = jnp.exp(s) * mask
            dens.append(jnp.sum(p, axis=1, keepdims=True))
            nums.append(jax.lax.dot_general(
                p, vh[:, h * DH:(h + 1) * DH],
                (((1,), (0,)), ((), ())), preferred_element_type=jnp.float32))
        return (num + jnp.concatenate(nums, axis=1),
                den + jnp.concatenate(dens, axis=1))

    num0 = jnp.zeros((BQ, TD), jnp.float32)
    den0 = jnp.zeros((BQ, H), jnp.float32)
    num, den = jax.lax.fori_loop(c0, c1, body, (num0, den0))
    num_ref[...] = num
    den_ref[...] = den


def _combine_body(q_ref, na_ref, nb_ref, da_ref, db_ref, m_ref,
                  wg_ref, bg_ref, wo_ref, bo_ref, out_ref):
    m = m_ref[...][:, None]
    num = na_ref[...] + nb_ref[...] * m
    den = da_ref[...] + db_ref[...] * m
    parts = [num[:, h * DH:(h + 1) * DH] / den[:, h:h + 1] for h in range(H)]
    o = jnp.concatenate(parts, axis=1)
    g = jax.lax.dot_general(
        q_ref[...], wg_ref[...], (((1,), (1,)), ((), ())),
        preferred_element_type=jnp.float32) + bg_ref[...]
    o = o / (1.0 + jnp.exp(-g))
    out_ref[...] = jax.lax.dot_general(
        o, wo_ref[...], (((1,), (1,)), ((), ())),
        preferred_element_type=jnp.float32) + bo_ref[...]


def kernel(q, k, v, edge_index, Wq, Wk, Wv, Wo, bo, Wg, bg):
    f32 = jnp.float32
    src = edge_index[:, 0]
    dst = edge_index[:, 1]
    perm_s = jnp.argsort(src)
    perm_d = jnp.argsort(dst)
    srcs = jnp.take(src, perm_s)
    dstd = jnp.take(dst, perm_d)
    counts = jnp.bincount(src, length=N)
    off = jnp.concatenate([jnp.zeros((1,), jnp.int32),
                           jnp.cumsum(counts).astype(jnp.int32)])

    qcat = _sc_gather_rows(q, jnp.concatenate([perm_s, perm_d]), 200)
    qnode_i = jnp.concatenate([srcs, dstd]).astype(jnp.int32)
    perm_sp = jnp.concatenate([perm_s, jnp.zeros((EK - E,), perm_s.dtype)])
    kps = _sc_gather_rows(k, perm_sp, 168)
    vps = _sc_gather_rows(v, perm_sp, 168)
    kn = jnp.concatenate([srcs.astype(f32), jnp.full((EK - E,), -1.0, f32)])

    starts = jnp.take(off, qnode_i[0::BQ]).astype(jnp.int32)
    ends = jnp.take(off, qnode_i[BQ - 1::BQ] + 1).astype(jnp.int32)
    qn = qnode_i.astype(f32)

    qh = pl.pallas_call(
        _proj_q_body,
        grid=(NBQ,),
        in_specs=[pl.BlockSpec((BQ, D), lambda b: (b, 0)),
                  pl.BlockSpec((TD, D), lambda b: (0, 0))],
        out_specs=pl.BlockSpec((BQ, TD), lambda b: (b, 0)),
        out_shape=jax.ShapeDtypeStruct((2 * E, TD), f32),
        interpret=_INTERPRET,
    )(qcat, Wq)

    khs, vhs = pl.pallas_call(
        _proj_kv_body,
        grid=(EK // BQ,),
        in_specs=[pl.BlockSpec((BQ, D), lambda b: (b, 0)),
                  pl.BlockSpec((BQ, D), lambda b: (b, 0)),
                  pl.BlockSpec((TD, D), lambda b: (0, 0)),
                  pl.BlockSpec((TD, D), lambda b: (0, 0))],
        out_specs=[pl.BlockSpec((BQ, TD), lambda b: (b, 0)),
                   pl.BlockSpec((BQ, TD), lambda b: (b, 0))],
        out_shape=[jax.ShapeDtypeStruct((EK, TD), f32),
                   jax.ShapeDtypeStruct((EK, TD), f32)],
        interpret=_INTERPRET,
    )(kps, vps, Wk, Wv)

    numer, den = pl.pallas_call(
        _attn_body,
        grid=(NBQ,),
        in_specs=[pl.BlockSpec(memory_space=pltpu.SMEM),
                  pl.BlockSpec(memory_space=pltpu.SMEM),
                  pl.BlockSpec((BQ, TD), lambda b: (b, 0)),
                  pl.BlockSpec((BQ,), lambda b: (b,)),
                  pl.BlockSpec((EK, TD), lambda b: (0, 0)),
                  pl.BlockSpec((EK, TD), lambda b: (0, 0)),
                  pl.BlockSpec((EK,), lambda b: (0,))],
        out_specs=[pl.BlockSpec((BQ, TD), lambda b: (b, 0)),
                   pl.BlockSpec((BQ, H), lambda b: (b, 0))],
        out_shape=[jax.ShapeDtypeStruct((2 * E, TD), f32),
                   jax.ShapeDtypeStruct((2 * E, H), f32)],
        interpret=_INTERPRET,
    )(starts, ends, qh, qn, khs, vhs, kn)

    inv_s = jnp.zeros((E,), jnp.int32).at[perm_s].set(jnp.arange(E, dtype=jnp.int32))
    inv_d = jnp.zeros((E,), jnp.int32).at[perm_d].set(jnp.arange(E, dtype=jnp.int32))
    na = jnp.take(numer[:E], inv_s, axis=0)
    da = jnp.take(den[:E], inv_s, axis=0)
    nb = jnp.take(numer[E:], inv_d, axis=0)
    db = jnp.take(den[E:], inv_d, axis=0)
    m = (src != dst).astype(f32)

    out = pl.pallas_call(
        _combine_body,
        grid=(E // BQ,),
        in_specs=[pl.BlockSpec((BQ, D), lambda b: (b, 0)),
                  pl.BlockSpec((BQ, TD), lambda b: (b, 0)),
                  pl.BlockSpec((BQ, TD), lambda b: (b, 0)),
                  pl.BlockSpec((BQ, H), lambda b: (b, 0)),
                  pl.BlockSpec((BQ, H), lambda b: (b, 0)),
                  pl.BlockSpec((BQ,), lambda b: (b,)),
                  pl.BlockSpec((TD, D), lambda b: (0, 0)),
                  pl.BlockSpec((1, TD), lambda b: (0, 0)),
                  pl.BlockSpec((D, TD), lambda b: (0, 0)),
                  pl.BlockSpec((1, D), lambda b: (0, 0))],
        out_specs=pl.BlockSpec((BQ, D), lambda b: (b, 0)),
        out_shape=jax.ShapeDtypeStruct((E, D), f32),
        interpret=_INTERPRET,
    )(q, na, nb, da, db, m, Wg, bg.reshape(1, TD), Wo, bo.reshape(1, D))
    return out
